# baseline (speedup 1.0000x reference)
import jax
import jax.numpy as jnp
from jax import lax
from jax.experimental import pallas as pl
from jax.experimental.pallas import tpu as pltpu
from jax.experimental.pallas import tpu_sc as plsc

N = 10000
NP = 10240
D = 256
DH = 128
H = 512
O = 256
E = 160000
CH = 64
NB = 4
EP = 163840
NCHUNK = EP // CH
ROWS_PS = NCHUNK // 16
CROWS_PS = NCHUNK // 32
RPS = NP // 16
BN = 400


def _sc_body(x0, x1, srcl, dstl, out0, out1, cnt0, cnt1,
             src0, src1, src2, src3, dst0, dst1, dst2, dst3,
             rows0, rows1, rows2, rows3, acc_s,
             gs0, gs1, gs2, gs3, ss0, ss1, ss2, ss3, is0, is1, is2, is3):
    c = lax.axis_index("c")
    s = lax.axis_index("s")

    srcb = (src0, src1, src2, src3)
    dstb = (dst0, dst1, dst2, dst3)
    rows = (rows0, rows1, rows2, rows3)
    gsem = (gs0, gs1, gs2, gs3)
    ssem = (ss0, ss1, ss2, ss3)
    isem = (is0, is1, is2, is3)

    zeros16 = jnp.zeros((16,), jnp.float32)
    ones16 = jnp.ones((16,), jnp.float32)

    def fill_const(ref, v16):
        def frow(i, _):
            def fcol(j, _):
                ref[i, pl.ds(j * 16, 16)] = v16
                return 0
            lax.fori_loop(0, DH // 16, fcol, 0)
            return 0
        lax.fori_loop(0, CH, frow, 0)

    def zero_acc(zsrc_v):
        def z(k, _):
            pltpu.sync_copy(zsrc_v, acc_s.at[pl.ds(s * RPS + k * CH, CH)])
            return 0
        lax.fori_loop(0, RPS // CH, z, 0)

    def copy_acc(out_hbm, stage_v):
        def cp(k, _):
            r0 = s * RPS + k * CH
            pltpu.sync_copy(acc_s.at[pl.ds(r0, CH)], stage_v)
            pltpu.sync_copy(stage_v, out_hbm.at[pl.ds(r0, CH)])
            return 0
        lax.fori_loop(0, RPS // CH, cp, 0)

    fill_const(rows0, zeros16)
    zero_acc(rows0)
    plsc.subcore_barrier()

    def idx_load(off, q):
        a = pltpu.async_copy(srcl.at[pl.ds(off, CH)], srcb[q], isem[q])
        b = pltpu.async_copy(dstl.at[pl.ds(off, CH)], dstb[q], isem[q])
        return a, b

    def feat_loop(x_hbm):
        base = s * ROWS_PS * CH
        last = base + (ROWS_PS - NB) * CH

        ii = [idx_load(base + q * CH, q) for q in range(NB)]
        gg = []
        for q in range(NB):
            ii[q][0].wait(); ii[q][1].wait()
            gg.append(pltpu.async_copy(x_hbm.at[srcb[q]], rows[q], gsem[q]))

        def it(bb, _):
            offn = jnp.minimum(base + (NB * bb + NB) * CH, last)
            sc = []
            for q in range(NB):
                gg[q].wait()
                sc.append(pltpu.async_copy(rows[q], acc_s.at[dstb[q]],
                                           ssem[q], add=True))
            jj = []
            for q in range(NB):
                sc[q].wait()
                jj.append(idx_load(offn + q * CH, q))
            for q in range(NB):
                jj[q][0].wait(); jj[q][1].wait()
                pltpu.async_copy(x_hbm.at[srcb[q]], rows[q], gsem[q])
            return 0
        lax.fori_loop(0, ROWS_PS // NB, it, 0)
        for q in range(NB):
            gg[q].wait()

    pl.when(c == 0)(lambda: feat_loop(x0))
    pl.when(c == 1)(lambda: feat_loop(x1))

    plsc.subcore_barrier()
    pl.when(c == 0)(lambda: copy_acc(out0, rows0))
    pl.when(c == 1)(lambda: copy_acc(out1, rows0))
    plsc.subcore_barrier()

    fill_const(rows1, zeros16)
    zero_acc(rows1)
    fill_const(rows0, ones16)
    plsc.subcore_barrier()

    def cnt_loop(_=None):
        base = (c * 16 + s) * CROWS_PS * CH
        last = base + (CROWS_PS - NB) * CH
        for q in range(NB):
            pltpu.sync_copy(dstl.at[pl.ds(base + q * CH, CH)], dstb[q])

        def it(bb, _):
            offn = jnp.minimum(base + (NB * bb + NB) * CH, last)
            sc = [pltpu.async_copy(rows0, acc_s.at[dstb[q]], ssem[q],
                                   add=True) for q in range(NB)]
            for q in range(NB):
                sc[q].wait()
                pltpu.sync_copy(dstl.at[pl.ds(offn + q * CH, CH)], dstb[q])
            return 0
        lax.fori_loop(0, CROWS_PS // NB, it, 0)

    cnt_loop()
    plsc.subcore_barrier()
    pl.when(c == 0)(lambda: copy_acc(cnt0, rows1))
    pl.when(c == 1)(lambda: copy_acc(cnt1, rows1))


def _sc_aggregate(x0, x1, srcl, dstl):
    fn = pl.kernel(
        _sc_body,
        out_type=[
            jax.ShapeDtypeStruct((NP, DH), jnp.float32),
            jax.ShapeDtypeStruct((NP, DH), jnp.float32),
            jax.ShapeDtypeStruct((NP, DH), jnp.float32),
            jax.ShapeDtypeStruct((NP, DH), jnp.float32),
        ],
        mesh=plsc.VectorSubcoreMesh(core_axis_name="c", subcore_axis_name="s"),
        scratch_types=(
            [pltpu.VMEM((CH,), jnp.int32) for _ in range(8)]
            + [pltpu.VMEM((CH, DH), jnp.float32) for _ in range(4)]
            + [pltpu.VMEM_SHARED((NP, DH), jnp.float32)]
            + [pltpu.SemaphoreType.DMA for _ in range(12)]
        ),
    )
    return fn(x0, x1, srcl, dstl)


def _tc_body(s0, s1, c0, c1, x, wl, bl, wr, w1, b1, w2, b2, out):
    f32 = jnp.float32
    inv = 1.0 / jnp.maximum(c0[:, 0:1] + c1[:, 0:1], 1.0)
    mean0 = s0[...] * inv
    mean1 = s1[...] * inv
    h = (jnp.dot(mean0, wl[0:DH, :], preferred_element_type=f32)
         + jnp.dot(mean1, wl[DH:D, :], preferred_element_type=f32)
         + jnp.dot(x[...], wr[...], preferred_element_type=f32)
         + bl[...])
    t = jnp.maximum(jnp.dot(h, w1[...], preferred_element_type=f32) + b1[...], 0.0)
    out[...] = jnp.dot(t, w2[...], preferred_element_type=f32) + b2[...]


def _tc_dense(s0, s1, c0, c1, x, W_l, b_l, W_r, W1, b1, W2, b2):
    grid = (N // BN,)
    full = lambda shape: pl.BlockSpec(shape, lambda i: (0, 0))
    blk = lambda w: pl.BlockSpec((BN, w), lambda i: (i, 0))
    return pl.pallas_call(
        _tc_body,
        grid=grid,
        in_specs=[
            blk(DH), blk(DH), blk(DH), blk(DH), blk(D),
            full((D, H)), full((1, H)), full((D, H)),
            full((H, H)), full((1, H)), full((H, O)), full((1, O)),
        ],
        out_specs=blk(O),
        out_shape=jax.ShapeDtypeStruct((N, O), jnp.float32),
        compiler_params=pltpu.CompilerParams(
            dimension_semantics=("parallel",)),
    )(s0, s1, c0, c1, x, W_l, b_l, W_r, W1, b1, W2, b2)


def kernel(x, edge_index, W_l, b_l, W_r, W1, b1, W2, b2):
    src = edge_index[0].astype(jnp.int32)
    dst = edge_index[1].astype(jnp.int32)
    srcl = jnp.concatenate([src, jnp.zeros((EP - E,), jnp.int32)])
    dstl = jnp.concatenate([dst, jnp.full((EP - E,), NP - 1, jnp.int32)])
    x0 = x[:, :DH]
    x1 = x[:, DH:]
    s0, s1, c0, c1 = _sc_aggregate(x0, x1, srcl, dstl)
    return _tc_dense(s0, s1, c0, c1, x,
                     W_l, b_l.reshape(1, H), W_r,
                     W1, b1.reshape(1, H), W2, b2.reshape(1, O))
